# Initial kernel scaffold; baseline (speedup 1.0000x reference)
import jax
import jax.numpy as jnp
from jax.experimental import pallas as pl


def kernel(emb_user, emb_item, adj):
    raise NotImplementedError("write your pallas kernel here")


# speedup vs baseline: 9.1253x; 9.1253x over previous
import functools

import jax
import jax.numpy as jnp
from jax.experimental import pallas as pl
from jax.experimental.pallas import tpu as pltpu

N = 4096
EMB = 64
TOP_H = 20
ALPHA = 0.5
BETA = 0.5

RB = 512
NB = N // RB
RT = 256
NT = N // RT


def _gcn_kernel(a_ref, u0_ref, i0_ref, up_ref, ip_ref, nu_ref, ni_ref, acc_ref):
    i = pl.program_id(0)
    u0 = u0_ref[...]
    i0 = i0_ref[...]
    sim = jax.nn.sigmoid(
        jax.lax.dot_general(u0, i0, (((1,), (1,)), ((), ())),
                            preferred_element_type=jnp.float32))
    A = a_ref[...] * (0.5 * sim + 0.5)
    nu_ref[...] = jax.lax.dot_general(
        A, ip_ref[...], (((1,), (0,)), ((), ())),
        preferred_element_type=jnp.float32)
    contrib = jax.lax.dot_general(
        A, up_ref[...], (((0,), (0,)), ((), ())),
        preferred_element_type=jnp.float32)

    @pl.when(i == 0)
    def _():
        acc_ref[...] = contrib

    @pl.when(i > 0)
    def _():
        acc_ref[...] = acc_ref[...] + contrib

    @pl.when(i == NB - 1)
    def _():
        ni_ref[...] = acc_ref[...]


def _gcn_pass(adj, u0, i0, uprev, iprev):
    return pl.pallas_call(
        _gcn_kernel,
        grid=(NB,),
        in_specs=[
            pl.BlockSpec((RB, N), lambda i: (i, 0)),
            pl.BlockSpec((RB, EMB), lambda i: (i, 0)),
            pl.BlockSpec((N, EMB), lambda i: (0, 0)),
            pl.BlockSpec((RB, EMB), lambda i: (i, 0)),
            pl.BlockSpec((N, EMB), lambda i: (0, 0)),
        ],
        out_specs=[
            pl.BlockSpec((RB, EMB), lambda i: (i, 0)),
            pl.BlockSpec((N, EMB), lambda i: (0, 0)),
        ],
        out_shape=[
            jax.ShapeDtypeStruct((N, EMB), jnp.float32),
            jax.ShapeDtypeStruct((N, EMB), jnp.float32),
        ],
        scratch_shapes=[pltpu.VMEM((N, EMB), jnp.float32)],
    )(adj, u0, i0, uprev, iprev)


def _topsim_kernel(x_ref, xb_ref, tbl_ref, e0_ref, e1_ref, out_ref):
    x = x_ref[...]
    mu = jnp.mean(x)
    xc = x - mu
    xb = xb_ref[...] - mu
    rn_b = 1.0 / (jnp.sqrt(jnp.sum(xb * xb, axis=1, keepdims=True)) + 1e-8)
    sq = xc * xc
    cn = jax.lax.dot_general(
        jnp.ones((1, EMB), jnp.float32), sq, (((1,), (1,)), ((), ())),
        preferred_element_type=jnp.float32)
    rn_c = 1.0 / (jnp.sqrt(cn) + 1e-8)
    sim = jax.lax.dot_general(
        xb, xc, (((1,), (1,)), ((), ())),
        preferred_element_type=jnp.float32) * rn_b * rn_c

    colio = jax.lax.broadcasted_iota(jnp.int32, (RT, N), 1)
    rem = sim
    v1 = None
    v21 = None
    for t in range(TOP_H + 1):
        m = jnp.max(rem, axis=1, keepdims=True)
        if t == 0:
            v1 = m
        if t == TOP_H:
            v21 = m
        else:
            cand = jnp.where(rem == m, colio, N)
            first = jnp.min(cand, axis=1, keepdims=True)
            rem = jnp.where(colio == first, -jnp.inf, rem)

    keep = jnp.logical_and(sim >= v21, sim < v1)
    masked = jnp.where(keep, sim, 0.0)
    semb = jax.lax.dot_general(
        masked, tbl_ref[...], (((1,), (0,)), ((), ())),
        preferred_element_type=jnp.float32) * (1.0 / TOP_H)
    out_ref[...] = (e0_ref[...] + e1_ref[...] + xb_ref[...]) * (1.0 / 3.0) \
        + ALPHA * semb


def _topsim_pass(x, tbl, e1):
    return pl.pallas_call(
        _topsim_kernel,
        grid=(NT,),
        in_specs=[
            pl.BlockSpec((N, EMB), lambda i: (0, 0)),
            pl.BlockSpec((RT, EMB), lambda i: (i, 0)),
            pl.BlockSpec((N, EMB), lambda i: (0, 0)),
            pl.BlockSpec((RT, EMB), lambda i: (i, 0)),
            pl.BlockSpec((RT, EMB), lambda i: (i, 0)),
        ],
        out_specs=pl.BlockSpec((RT, EMB), lambda i: (i, 0)),
        out_shape=jax.ShapeDtypeStruct((N, EMB), jnp.float32),
    )(x, x, tbl, tbl, e1)


def kernel(emb_user, emb_item, adj):
    u1, i1 = _gcn_pass(adj, emb_user, emb_item, emb_user, emb_item)
    u2, i2 = _gcn_pass(adj, emb_user, emb_item, u1, i1)
    u_out = _topsim_pass(u2, emb_user, u1)
    i_out = _topsim_pass(i2, emb_item, i1)
    return u_out, i_out


# speedup vs baseline: 17.1270x; 1.8769x over previous
import functools

import jax
import jax.numpy as jnp
from jax.experimental import pallas as pl
from jax.experimental.pallas import tpu as pltpu

N = 4096
EMB = 64
TOP_H = 20
ALPHA = 0.5
BETA = 0.5

RB = 512
NB = N // RB
RT = 256
NT = N // RT


def _gcn_kernel(a_ref, u0_ref, i0_ref, up_ref, ip_ref, nu_ref, ni_ref, acc_ref):
    i = pl.program_id(0)
    u0 = u0_ref[...]
    i0 = i0_ref[...]
    sim = jax.nn.sigmoid(
        jax.lax.dot_general(u0, i0, (((1,), (1,)), ((), ())),
                            preferred_element_type=jnp.float32))
    A = a_ref[...] * (0.5 * sim + 0.5)
    nu_ref[...] = jax.lax.dot_general(
        A, ip_ref[...], (((1,), (0,)), ((), ())),
        preferred_element_type=jnp.float32)
    contrib = jax.lax.dot_general(
        A, up_ref[...], (((0,), (0,)), ((), ())),
        preferred_element_type=jnp.float32)

    @pl.when(i == 0)
    def _():
        acc_ref[...] = contrib

    @pl.when(i > 0)
    def _():
        acc_ref[...] = acc_ref[...] + contrib

    @pl.when(i == NB - 1)
    def _():
        ni_ref[...] = acc_ref[...]


def _gcn_pass(adj, u0, i0, uprev, iprev):
    return pl.pallas_call(
        _gcn_kernel,
        grid=(NB,),
        in_specs=[
            pl.BlockSpec((RB, N), lambda i: (i, 0)),
            pl.BlockSpec((RB, EMB), lambda i: (i, 0)),
            pl.BlockSpec((N, EMB), lambda i: (0, 0)),
            pl.BlockSpec((RB, EMB), lambda i: (i, 0)),
            pl.BlockSpec((N, EMB), lambda i: (0, 0)),
        ],
        out_specs=[
            pl.BlockSpec((RB, EMB), lambda i: (i, 0)),
            pl.BlockSpec((N, EMB), lambda i: (0, 0)),
        ],
        out_shape=[
            jax.ShapeDtypeStruct((N, EMB), jnp.float32),
            jax.ShapeDtypeStruct((N, EMB), jnp.float32),
        ],
        scratch_shapes=[pltpu.VMEM((N, EMB), jnp.float32)],
    )(adj, u0, i0, uprev, iprev)


def _topsim_kernel(x_ref, xb_ref, tbl_ref, e0_ref, e1_ref, out_ref):
    x = x_ref[...]
    mu = jnp.mean(x)
    xc = x - mu
    xb = xb_ref[...] - mu
    rn_b = 1.0 / (jnp.sqrt(jnp.sum(xb * xb, axis=1, keepdims=True)) + 1e-8)
    sq = xc * xc
    cn = jax.lax.dot_general(
        jnp.ones((1, EMB), jnp.float32), sq, (((1,), (1,)), ((), ())),
        preferred_element_type=jnp.float32)
    rn_c = 1.0 / (jnp.sqrt(cn) + 1e-8)
    sim = jax.lax.dot_general(
        xb, xc, (((1,), (1,)), ((), ())),
        preferred_element_type=jnp.float32) * rn_b * rn_c

    rem = sim
    v1 = None
    v21 = None
    for t in range(TOP_H + 1):
        m = jnp.max(rem, axis=1, keepdims=True)
        if t == 0:
            v1 = m
        if t == TOP_H:
            v21 = m
        else:
            rem = jnp.where(rem >= m, -jnp.inf, rem)

    keep = jnp.logical_and(sim >= v21, sim < v1)
    masked = jnp.where(keep, sim, 0.0)
    semb = jax.lax.dot_general(
        masked, tbl_ref[...], (((1,), (0,)), ((), ())),
        preferred_element_type=jnp.float32) * (1.0 / TOP_H)
    out_ref[...] = (e0_ref[...] + e1_ref[...] + xb_ref[...]) * (1.0 / 3.0) \
        + ALPHA * semb


def _topsim_pass(x, tbl, e1):
    return pl.pallas_call(
        _topsim_kernel,
        grid=(NT,),
        in_specs=[
            pl.BlockSpec((N, EMB), lambda i: (0, 0)),
            pl.BlockSpec((RT, EMB), lambda i: (i, 0)),
            pl.BlockSpec((N, EMB), lambda i: (0, 0)),
            pl.BlockSpec((RT, EMB), lambda i: (i, 0)),
            pl.BlockSpec((RT, EMB), lambda i: (i, 0)),
        ],
        out_specs=pl.BlockSpec((RT, EMB), lambda i: (i, 0)),
        out_shape=jax.ShapeDtypeStruct((N, EMB), jnp.float32),
    )(x, x, tbl, tbl, e1)


def kernel(emb_user, emb_item, adj):
    u1, i1 = _gcn_pass(adj, emb_user, emb_item, emb_user, emb_item)
    u2, i2 = _gcn_pass(adj, emb_user, emb_item, u1, i1)
    u_out = _topsim_pass(u2, emb_user, u1)
    i_out = _topsim_pass(i2, emb_item, i1)
    return u_out, i_out


# speedup vs baseline: 23.1709x; 1.3529x over previous
import functools

import jax
import jax.numpy as jnp
from jax.experimental import pallas as pl
from jax.experimental.pallas import tpu as pltpu

N = 4096
EMB = 64
TOP_H = 20
ALPHA = 0.5
BETA = 0.5

RB = 512
NB = N // RB
RT = 256
NT = N // RT


def _gcn_kernel(a_ref, u0_ref, i0_ref, up_ref, ip_ref, nu_ref, ni_ref, acc_ref):
    i = pl.program_id(0)
    u0 = u0_ref[...]
    i0 = i0_ref[...]
    sim = jax.nn.sigmoid(
        jax.lax.dot_general(u0, i0, (((1,), (1,)), ((), ())),
                            preferred_element_type=jnp.float32))
    A = a_ref[...] * (0.5 * sim + 0.5)
    nu_ref[...] = jax.lax.dot_general(
        A, ip_ref[...], (((1,), (0,)), ((), ())),
        preferred_element_type=jnp.float32)
    contrib = jax.lax.dot_general(
        A, up_ref[...], (((0,), (0,)), ((), ())),
        preferred_element_type=jnp.float32)

    @pl.when(i == 0)
    def _():
        acc_ref[...] = contrib

    @pl.when(i > 0)
    def _():
        acc_ref[...] = acc_ref[...] + contrib

    @pl.when(i == NB - 1)
    def _():
        ni_ref[...] = acc_ref[...]


def _gcn_pass(adj, u0, i0, uprev, iprev):
    return pl.pallas_call(
        _gcn_kernel,
        grid=(NB,),
        in_specs=[
            pl.BlockSpec((RB, N), lambda i: (i, 0)),
            pl.BlockSpec((RB, EMB), lambda i: (i, 0)),
            pl.BlockSpec((N, EMB), lambda i: (0, 0)),
            pl.BlockSpec((RB, EMB), lambda i: (i, 0)),
            pl.BlockSpec((N, EMB), lambda i: (0, 0)),
        ],
        out_specs=[
            pl.BlockSpec((RB, EMB), lambda i: (i, 0)),
            pl.BlockSpec((N, EMB), lambda i: (0, 0)),
        ],
        out_shape=[
            jax.ShapeDtypeStruct((N, EMB), jnp.float32),
            jax.ShapeDtypeStruct((N, EMB), jnp.float32),
        ],
        scratch_shapes=[pltpu.VMEM((N, EMB), jnp.float32)],
    )(adj, u0, i0, uprev, iprev)


def _topsim_kernel(x_ref, xb_ref, tbl_ref, e0_ref, e1_ref, out_ref):
    x = x_ref[...]
    mu = jnp.mean(x)
    xc = x - mu
    xb = xb_ref[...] - mu
    rn_b = 1.0 / (jnp.sqrt(jnp.sum(xb * xb, axis=1, keepdims=True)) + 1e-8)
    sq = xc * xc
    cn = jax.lax.dot_general(
        jnp.ones((1, EMB), jnp.float32), sq, (((1,), (1,)), ((), ())),
        preferred_element_type=jnp.float32)
    rn_c = 1.0 / (jnp.sqrt(cn) + 1e-8)
    sim = jax.lax.dot_general(
        xb, xc, (((1,), (1,)), ((), ())),
        preferred_element_type=jnp.float32) * rn_b * rn_c

    sb = sim.astype(jnp.bfloat16)
    neg = jnp.array(-jnp.inf, jnp.bfloat16)
    m = jnp.max(sb, axis=1, keepdims=True)
    v1 = m
    for _ in range(TOP_H):
        m = jnp.max(jnp.where(sb < m, sb, neg), axis=1, keepdims=True)
    v21 = m

    keep = jnp.logical_and(sb >= v21, sb < v1)
    masked = jnp.where(keep, sim, 0.0)
    semb = jax.lax.dot_general(
        masked, tbl_ref[...], (((1,), (0,)), ((), ())),
        preferred_element_type=jnp.float32) * (1.0 / TOP_H)
    out_ref[...] = (e0_ref[...] + e1_ref[...] + xb_ref[...]) * (1.0 / 3.0) \
        + ALPHA * semb


def _topsim_pass(x, tbl, e1):
    return pl.pallas_call(
        _topsim_kernel,
        grid=(NT,),
        in_specs=[
            pl.BlockSpec((N, EMB), lambda i: (0, 0)),
            pl.BlockSpec((RT, EMB), lambda i: (i, 0)),
            pl.BlockSpec((N, EMB), lambda i: (0, 0)),
            pl.BlockSpec((RT, EMB), lambda i: (i, 0)),
            pl.BlockSpec((RT, EMB), lambda i: (i, 0)),
        ],
        out_specs=pl.BlockSpec((RT, EMB), lambda i: (i, 0)),
        out_shape=jax.ShapeDtypeStruct((N, EMB), jnp.float32),
    )(x, x, tbl, tbl, e1)


def kernel(emb_user, emb_item, adj):
    u1, i1 = _gcn_pass(adj, emb_user, emb_item, emb_user, emb_item)
    u2, i2 = _gcn_pass(adj, emb_user, emb_item, u1, i1)
    u_out = _topsim_pass(u2, emb_user, u1)
    i_out = _topsim_pass(i2, emb_item, i1)
    return u_out, i_out
